# Initial kernel scaffold; baseline (speedup 1.0000x reference)
import jax
import jax.numpy as jnp
from jax.experimental import pallas as pl


def kernel(x, W1, b1, W2, b2, Wl, bl, layer_idx):
    raise NotImplementedError("write your pallas kernel here")


# speedup vs baseline: 1.2983x; 1.2983x over previous
import functools

import jax
import jax.numpy as jnp
from jax.experimental import pallas as pl
from jax.experimental.pallas import tpu as pltpu

_BM = 512


def _gating_block_kernel(idx_ref, x_ref, w1_ref, b1_ref, w2_ref, b2_ref,
                         wl_ref, bl_ref, out_ref, s1_ref):
    i = pl.program_id(0)
    num_layers = w2_ref.shape[1]

    @pl.when(i == 0)
    def _init_s1():
        s1_ref[...] = jnp.sum(w1_ref[...], axis=0, keepdims=True)

    x = x_ref[...]
    d = x.shape[1]
    m = jnp.mean(x, axis=1, keepdims=True)
    h = jax.nn.relu(m * s1_ref[...] + b1_ref[...])
    logits = jnp.dot(h, w2_ref[...],
                     preferred_element_type=jnp.float32) + b2_ref[...]
    onehot = jax.lax.broadcasted_iota(jnp.int32, (1, num_layers), 1) == idx_ref[0]
    logit = jnp.sum(jnp.where(onehot, logits, 0.0), axis=1, keepdims=True)
    gate = jax.nn.sigmoid(logit) > 0.5

    y = jnp.dot(x.astype(jnp.bfloat16), wl_ref[...],
                preferred_element_type=jnp.float32)
    y = jax.nn.relu(y + bl_ref[...])
    out_ref[...] = jnp.where(gate, y, x)


def kernel(x, W1, b1, W2, b2, Wl, bl, layer_idx):
    n, d = x.shape
    h_dim = W1.shape[1]
    n_layers = W2.shape[1]
    idx = jnp.asarray(layer_idx, jnp.int32).reshape((1,))
    wl_bf = Wl.astype(jnp.bfloat16)

    grid_spec = pltpu.PrefetchScalarGridSpec(
        num_scalar_prefetch=1,
        grid=(n // _BM,),
        in_specs=[
            pl.BlockSpec((_BM, d), lambda i, s: (i, 0)),
            pl.BlockSpec((d, h_dim), lambda i, s: (0, 0)),
            pl.BlockSpec((1, h_dim), lambda i, s: (0, 0)),
            pl.BlockSpec((h_dim, n_layers), lambda i, s: (0, 0)),
            pl.BlockSpec((1, n_layers), lambda i, s: (0, 0)),
            pl.BlockSpec((d, d), lambda i, s: (0, 0)),
            pl.BlockSpec((1, d), lambda i, s: (0, 0)),
        ],
        out_specs=pl.BlockSpec((_BM, d), lambda i, s: (i, 0)),
        scratch_shapes=[pltpu.VMEM((1, h_dim), jnp.float32)],
    )
    return pl.pallas_call(
        _gating_block_kernel,
        grid_spec=grid_spec,
        out_shape=jax.ShapeDtypeStruct((n, d), jnp.float32),
    )(idx, x, W1, b1.reshape(1, h_dim), W2, b2.reshape(1, n_layers),
      wl_bf, bl.reshape(1, d))
